# Initial kernel scaffold; baseline (speedup 1.0000x reference)
import jax
import jax.numpy as jnp
from jax.experimental import pallas as pl


def kernel(encoder_outputs, neighbor_indices, grid_positions, conv3x1_w, conv3x1_b, conv1x1_w, conv1x1_b, fc_w, fc_b):
    raise NotImplementedError("write your pallas kernel here")


# speedup vs baseline: 2.6480x; 2.6480x over previous
import functools

import jax
import jax.numpy as jnp
from jax import lax
from jax.experimental import pallas as pl
from jax.experimental.pallas import tpu as pltpu
from jax.experimental.pallas import tpu_sc as plsc

GX, GY = 15, 3
NROWS = GX * GY
CP = 48


def _leaky(v):
    return jnp.where(v >= 0, v, 0.1 * v)


def _premul_body(enc_ref, w_ref, out_ref):
    out_ref[...] = jnp.dot(enc_ref[...], w_ref[...],
                           preferred_element_type=jnp.float32)


def _make_sc_grid(num_agents, batch, k_nbr):
    info = plsc.get_sparse_core_info()
    nc, ns = info.num_cores, info.num_subcores
    nw = nc * ns
    spw = batch // nw
    grp = 8
    ngrp = spw // grp
    grows = grp * NROWS
    mesh = plsc.VectorSubcoreMesh(core_axis_name="c", subcore_axis_name="s")

    @functools.partial(
        pl.kernel,
        mesh=mesh,
        compiler_params=pltpu.CompilerParams(needs_layout_passes=False,
                                             use_tc_tiling_on_sc=False),
        out_type=jax.ShapeDtypeStruct((batch * NROWS, CP), jnp.float32),
        scratch_types=[
            pltpu.VMEM((2 * num_agents,), jnp.int32),
            pltpu.VMEM((spw, k_nbr), jnp.int32),
            pltpu.VMEM((k_nbr, CP), jnp.float32),
            pltpu.VMEM((grows + 1, CP), jnp.float32),
            pltpu.SemaphoreType.DMA,
        ],
    )
    def sc_grid(ze_hbm, nbr_hbm, pos_hbm, grid_hbm,
                pos_v, idx_v, ze_v, gbuf_v, sem):
        wid = lax.axis_index("s") * nc + lax.axis_index("c")
        base = wid * spw
        pltpu.sync_copy(pos_hbm, pos_v)
        pltpu.sync_copy(nbr_hbm.at[pl.ds(base, spw)], idx_v)
        zf = jnp.zeros((16,), jnp.float32)

        def zero_body(r, carry):
            for j in range(CP // 16):
                gbuf_v[r, pl.ds(j * 16, 16)] = zf
            return carry

        def body(g, carry):
            lax.fori_loop(0, grows + 1, zero_body, 0)
            for s8 in range(grp):
                i = g * grp + s8
                cp = pltpu.async_copy(ze_hbm.at[idx_v.at[i]], ze_v, sem)
                rows = []
                for h in range(k_nbr // 16):
                    idx16 = idx_v[i, pl.ds(h * 16, 16)]
                    x = plsc.load_gather(pos_v, [idx16 * 2])
                    y = plsc.load_gather(pos_v, [idx16 * 2 + 1])
                    rows.append(jnp.where(y < GY,
                                          s8 * NROWS + x * GY + y, grows))
                cp.wait()
                for h in range(k_nbr // 16):
                    for kk in range(16):
                        c = rows[h][kk]
                        k = h * 16 + kk
                        for j in range(CP // 16):
                            gbuf_v[c, pl.ds(j * 16, 16)] = (
                                ze_v[k, pl.ds(j * 16, 16)])
            pltpu.sync_copy(gbuf_v.at[pl.ds(0, grows)],
                            grid_hbm.at[pl.ds((base + g * grp) * NROWS,
                                              grows)])
            return carry

        lax.fori_loop(0, ngrp, body, 0)

    return sc_grid


def _head_body(g_ref, w1_ref, b3_ref, b1_ref, fcw_ref, fcb_ref, out_ref):
    Z = g_ref[...]
    n = Z.shape[0]
    Za, Zb, Zc = Z[:, 0:16], Z[:, 16:32], Z[:, 32:48]
    pad = jnp.zeros((3, 16), jnp.float32)
    Za_sh = jnp.concatenate([pad, Za[:-3]], axis=0)
    Zc_sh = jnp.concatenate([Zc[3:], pad], axis=0)
    rl = lax.broadcasted_iota(jnp.int32, (n, 16), 0) % NROWS
    H1 = (Zb + jnp.where(rl >= 3, Za_sh, 0.0)
          + jnp.where(rl < NROWS - 3, Zc_sh, 0.0) + b3_ref[...])
    H1 = _leaky(H1)
    H2 = _leaky(jnp.dot(H1, w1_ref[...],
                        preferred_element_type=jnp.float32) + b1_ref[...])
    M = jnp.max(H2.reshape(n // 9, 9, 32), axis=1)
    ns = n // NROWS
    si = lax.broadcasted_iota(jnp.int32, (ns, 5 * ns), 0)
    ti = lax.broadcasted_iota(jnp.int32, (ns, 5 * ns), 1)
    F = jnp.concatenate(
        [jnp.dot(jnp.where(ti == 5 * si + p, 1.0, 0.0), M,
                 preferred_element_type=jnp.float32) for p in range(5)],
        axis=1)
    out_ref[...] = _leaky(jnp.dot(F, fcw_ref[...],
                                  preferred_element_type=jnp.float32)
                          + fcb_ref[...])


def kernel(encoder_outputs, neighbor_indices, grid_positions, conv3x1_w,
           conv3x1_b, conv1x1_w, conv1x1_b, fc_w, fc_b):
    na, C, T = encoder_outputs.shape
    batch, k_nbr = neighbor_indices.shape

    w3flat = jnp.transpose(conv3x1_w[:, :, :, 0], (1, 2, 0)).reshape(C, CP)
    wbig = (jnp.zeros((C, T, CP), jnp.float32)
            .at[:, T - 1, :].set(w3flat).reshape(C * T, CP))
    w1 = jnp.transpose(conv1x1_w[:, :, 0, 0], (1, 0))
    fcw2 = jnp.transpose(fc_w.reshape(64, 32, 5), (2, 1, 0)).reshape(160, 64)
    enc2d = encoder_outputs.reshape(na, C * T)

    rb = 2000
    ze = pl.pallas_call(
        _premul_body,
        grid=(na // rb,),
        in_specs=[pl.BlockSpec((rb, C * T), lambda i: (i, 0)),
                  pl.BlockSpec((C * T, CP), lambda i: (0, 0))],
        out_specs=pl.BlockSpec((rb, CP), lambda i: (i, 0)),
        out_shape=jax.ShapeDtypeStruct((na, CP), jnp.float32),
    )(enc2d, wbig)

    grid_flat = _make_sc_grid(na, batch, k_nbr)(
        ze, neighbor_indices, grid_positions.reshape(-1))

    sb = 128
    out = pl.pallas_call(
        _head_body,
        grid=(batch // sb,),
        in_specs=[pl.BlockSpec((sb * NROWS, CP), lambda i: (i, 0)),
                  pl.BlockSpec((16, 32), lambda i: (0, 0)),
                  pl.BlockSpec((1, 16), lambda i: (0, 0)),
                  pl.BlockSpec((1, 32), lambda i: (0, 0)),
                  pl.BlockSpec((160, 64), lambda i: (0, 0)),
                  pl.BlockSpec((1, 64), lambda i: (0, 0))],
        out_specs=pl.BlockSpec((sb, 64), lambda i: (i, 0)),
        out_shape=jax.ShapeDtypeStruct((batch, 64), jnp.float32),
    )(grid_flat, w1, conv3x1_b.reshape(1, 16), conv1x1_b.reshape(1, 32),
      fcw2, fc_b.reshape(1, 64))
    return out


# speedup vs baseline: 3.4257x; 1.2937x over previous
import functools

import jax
import jax.numpy as jnp
import numpy as np
from jax import lax
from jax.experimental import pallas as pl
from jax.experimental.pallas import tpu as pltpu
from jax.experimental.pallas import tpu_sc as plsc

GX, GY = 15, 3
NROWS = GX * GY
CPAD = 128
PLANE = 5248
GROWS = 40


def _leaky(v):
    return jnp.where(v >= 0, v, 0.1 * v)


def _premul_body(enc_ref, w_ref, out_ref):
    out_ref[...] = jnp.dot(enc_ref[...], w_ref[...],
                           preferred_element_type=jnp.float32)


def _make_sc_grid(num_agents, batch, k_nbr):
    info = plsc.get_sparse_core_info()
    nc, ns = info.num_cores, info.num_subcores
    nw = nc * ns
    spw = batch // nw
    grp = 8
    ngrp = spw // grp
    npos = 2 * num_agents
    pchunk = npos // 5
    out_words = batch * 5 * 512
    mesh = plsc.VectorSubcoreMesh(core_axis_name="c", subcore_axis_name="s")

    @functools.partial(
        pl.kernel,
        mesh=mesh,
        compiler_params=pltpu.CompilerParams(needs_layout_passes=False,
                                             use_tc_tiling_on_sc=False),
        out_type=jax.ShapeDtypeStruct((out_words,), jnp.float32),
        scratch_types=[
            pltpu.VMEM((num_agents,), jnp.int32),
            pltpu.VMEM((pchunk,), jnp.int32),
            pltpu.VMEM((spw * k_nbr,), jnp.int32),
            pltpu.VMEM((2, k_nbr, CPAD), jnp.float32),
            pltpu.VMEM((4 * PLANE,), jnp.float32),
            pltpu.SemaphoreType.DMA,
            pltpu.SemaphoreType.DMA,
            pltpu.SemaphoreType.DMA,
        ],
    )
    def sc_grid(ze_hbm, nbr_hbm, pos_hbm, out_hbm,
                cell_v, chunk_v, idx_v, ze_v, gbuf_v, sem0, sem1, osem):
        wid = lax.axis_index("s") * nc + lax.axis_index("c")
        base = wid * spw
        pltpu.sync_copy(nbr_hbm.at[pl.ds(base * k_nbr, spw * k_nbr)], idx_v)
        lanes = lax.iota(jnp.int32, 16)
        zf = jnp.zeros((16,), jnp.float32)

        def cell_chunk(ci, carry):
            pltpu.sync_copy(pos_hbm.at[pl.ds(ci * pchunk, pchunk)], chunk_v)

            def cell_vec(v, carry2):
                r16 = v * 16 + lanes
                x = plsc.load_gather(chunk_v, [r16 * 2])
                y = plsc.load_gather(chunk_v, [r16 * 2 + 1])
                cell_v[pl.ds(ci * (pchunk // 2) + v * 16, 16)] = jnp.where(
                    y < GY, x * GY + y, NROWS)
                return carry2

            return lax.fori_loop(0, pchunk // 32, cell_vec, carry)

        lax.fori_loop(0, 5, cell_chunk, 0)

        def zero_vec(r, carry):
            gbuf_v[pl.ds(r * 16, 16)] = zf
            return carry

        def fire(i, buf):
            return pltpu.async_copy(
                ze_hbm.at[idx_v.at[pl.ds(i * k_nbr, k_nbr)]],
                ze_v.at[buf], [sem0, sem1][buf])

        def body(g, carry):
            cps = [fire(g * grp, 0)]
            @pl.when(g > 0)
            def _():
                for q in range(4):
                    pltpu.make_async_copy(
                        gbuf_v.at[pl.ds(q * PLANE, GROWS * 128)],
                        out_hbm.at[pl.ds(q * (out_words // 4), GROWS * 128)],
                        osem).wait()

            lax.fori_loop(0, 4 * PLANE // 16, zero_vec, 0)
            for s8 in range(grp):
                i = g * grp + s8
                if s8 + 1 < grp:
                    cps.append(fire(i + 1, (s8 + 1) % 2))
                cps[s8].wait()
                for h in range(k_nbr // 16):
                    idx16 = idx_v[pl.ds(i * k_nbr + h * 16, 16)]
                    c = plsc.load_gather(cell_v, [idx16])
                    for p, (dlt, lo, hi) in enumerate(
                            ((3, 0, NROWS - 4), (0, 0, NROWS - 1),
                             (-3, 3, NROWS - 1))):
                        r = c + dlt
                        t = (r * 456) >> 12
                        l512 = (r - 9 * t) * 48 + p * 16
                        q = l512 >> 7
                        row = jnp.where((c >= lo) & (c <= hi),
                                        s8 * 5 + t, GROWS)
                        off = q * PLANE + row * 128 + (l512 & 127)
                        for kk in range(16):
                            o = off[kk]
                            k = h * 16 + kk
                            gbuf_v[pl.ds(o, 16)] = (
                                ze_v[s8 % 2, k, pl.ds(p * 16, 16)])
            for q in range(4):
                pltpu.async_copy(
                    gbuf_v.at[pl.ds(q * PLANE, GROWS * 128)],
                    out_hbm.at[pl.ds(q * (out_words // 4)
                                     + (base + g * grp) * 5 * 128,
                                     GROWS * 128)], osem)
            return carry

        lax.fori_loop(0, ngrp, body, 0)
        for q in range(4):
            pltpu.make_async_copy(
                gbuf_v.at[pl.ds(q * PLANE, GROWS * 128)],
                out_hbm.at[pl.ds(q * (out_words // 4), GROWS * 128)],
                osem).wait()

    return sc_grid


def _head_body(g_ref, t_ref, w1b_ref, b3_ref, b1_ref, fcw_ref, fcb_ref,
               out_ref):
    H1 = jnp.dot(g_ref[0], t_ref[0], preferred_element_type=jnp.float32)
    for q in range(1, 4):
        H1 = H1 + jnp.dot(g_ref[q], t_ref[q],
                          preferred_element_type=jnp.float32)
    H1 = _leaky(H1 + b3_ref[...])
    H2 = _leaky(jnp.dot(H1, w1b_ref[...],
                        preferred_element_type=jnp.float32) + b1_ref[...])
    M = H2[:, 0:32]
    for j in range(1, 9):
        M = jnp.maximum(M, H2[:, 32 * j:32 * j + 32])
    nr = M.shape[0]
    ns = nr // 5
    si = lax.broadcasted_iota(jnp.int32, (ns, nr), 0)
    ti = lax.broadcasted_iota(jnp.int32, (ns, nr), 1)
    F = jnp.concatenate(
        [jnp.dot(jnp.where(ti == 5 * si + p, 1.0, 0.0), M,
                 preferred_element_type=jnp.float32) for p in range(5)],
        axis=1)
    out_ref[...] = _leaky(jnp.dot(F, fcw_ref[...],
                                  preferred_element_type=jnp.float32)
                          + fcb_ref[...])


def _tap_select():
    t = np.zeros((4, 128, 144), np.float32)
    for j in range(9):
        for p in range(3):
            for o in range(16):
                l512 = 48 * j + 16 * p + o
                t[l512 // 128, l512 % 128, 16 * j + o] = 1.0
    return jnp.asarray(t)


def kernel(encoder_outputs, neighbor_indices, grid_positions, conv3x1_w,
           conv3x1_b, conv1x1_w, conv1x1_b, fc_w, fc_b):
    na, C, T = encoder_outputs.shape
    batch, k_nbr = neighbor_indices.shape

    w3flat = jnp.transpose(conv3x1_w[:, :, :, 0], (1, 2, 0)).reshape(C, 48)
    wbig = (jnp.zeros((C, T, CPAD), jnp.float32)
            .at[:, T - 1, :48].set(w3flat).reshape(C * T, CPAD))
    w1 = jnp.transpose(conv1x1_w[:, :, 0, 0], (1, 0))
    w1b = jnp.zeros((144, 288), jnp.float32)
    for j in range(9):
        w1b = w1b.at[16 * j:16 * j + 16, 32 * j:32 * j + 32].set(w1)
    b3rep = jnp.tile(conv3x1_b, 9).reshape(1, 144)
    b1rep = jnp.tile(conv1x1_b, 9).reshape(1, 288)
    fcw2 = jnp.transpose(fc_w.reshape(64, 32, 5), (2, 1, 0)).reshape(160, 64)
    enc2d = encoder_outputs.reshape(na, C * T)

    rb = 2000
    ze = pl.pallas_call(
        _premul_body,
        grid=(na // rb,),
        in_specs=[pl.BlockSpec((rb, C * T), lambda i: (i, 0)),
                  pl.BlockSpec((C * T, CPAD), lambda i: (0, 0))],
        out_specs=pl.BlockSpec((rb, CPAD), lambda i: (i, 0)),
        out_shape=jax.ShapeDtypeStruct((na, CPAD), jnp.float32),
    )(enc2d, wbig)

    grid_flat = _make_sc_grid(na, batch, k_nbr)(
        ze, neighbor_indices.reshape(-1), grid_positions.reshape(-1))
    grid4 = grid_flat.reshape(4, batch * 5, 128)

    sb = 128
    out = pl.pallas_call(
        _head_body,
        grid=(batch // sb,),
        in_specs=[pl.BlockSpec((4, sb * 5, 128), lambda i: (0, i, 0)),
                  pl.BlockSpec((4, 128, 144), lambda i: (0, 0, 0)),
                  pl.BlockSpec((144, 288), lambda i: (0, 0)),
                  pl.BlockSpec((1, 144), lambda i: (0, 0)),
                  pl.BlockSpec((1, 288), lambda i: (0, 0)),
                  pl.BlockSpec((160, 64), lambda i: (0, 0)),
                  pl.BlockSpec((1, 64), lambda i: (0, 0))],
        out_specs=pl.BlockSpec((sb, 64), lambda i: (i, 0)),
        out_shape=jax.ShapeDtypeStruct((batch, 64), jnp.float32),
    )(grid4, _tap_select(), w1b, b3rep, b1rep, fcw2, fc_b.reshape(1, 64))
    return out


# speedup vs baseline: 3.5619x; 1.0397x over previous
import functools

import jax
import jax.numpy as jnp
import numpy as np
from jax import lax
from jax.experimental import pallas as pl
from jax.experimental.pallas import tpu as pltpu
from jax.experimental.pallas import tpu_sc as plsc

GX, GY = 15, 3
NROWS = GX * GY
CPAD = 128
PLANE = 5248
GROWS = 40


def _leaky(v):
    return jnp.where(v >= 0, v, 0.1 * v)


def _premul_body(enc_ref, w_ref, out_ref):
    out_ref[...] = jnp.dot(enc_ref[...], w_ref[...],
                           preferred_element_type=jnp.float32)


def _make_sc_grid(num_agents, batch, k_nbr):
    info = plsc.get_sparse_core_info()
    nc, ns = info.num_cores, info.num_subcores
    nw = nc * ns
    spw = batch // nw
    grp = 8
    ngrp = spw // grp
    npos = 2 * num_agents
    pchunk = npos // 5
    half = grp * k_nbr // 2
    out_words = batch * 5 * 512
    mesh = plsc.VectorSubcoreMesh(core_axis_name="c", subcore_axis_name="s")

    @functools.partial(
        pl.kernel,
        mesh=mesh,
        compiler_params=pltpu.CompilerParams(needs_layout_passes=False,
                                             use_tc_tiling_on_sc=False),
        out_type=jax.ShapeDtypeStruct((out_words,), jnp.float32),
        scratch_types=[
            pltpu.VMEM((num_agents,), jnp.int32),
            pltpu.VMEM((spw * k_nbr,), jnp.int32),
            pltpu.VMEM((grp * k_nbr, CPAD), jnp.float32),
            pltpu.VMEM((4 * PLANE,), jnp.float32),
            pltpu.SemaphoreType.DMA,
            pltpu.SemaphoreType.DMA,
            pltpu.SemaphoreType.DMA,
        ],
    )
    def sc_grid(ze_hbm, nbr_hbm, posf_hbm, out_hbm,
                cell_v, idx_v, ze_v, gbuf_v, sem0, sem1, osem):
        wid = lax.axis_index("s") * nc + lax.axis_index("c")
        base = wid * spw
        pltpu.sync_copy(nbr_hbm.at[pl.ds(base * k_nbr, spw * k_nbr)], idx_v)
        lanes = lax.iota(jnp.int32, 16)
        zf = jnp.zeros((16,), jnp.float32)

        def cell_chunk(ci, carry):
            pltpu.sync_copy(posf_hbm.at[pl.ds(ci * pchunk, pchunk)],
                            gbuf_v.at[pl.ds(0, pchunk)])

            def cell_vec(v, carry2):
                r16 = v * 16 + lanes
                x = plsc.bitcast(plsc.load_gather(gbuf_v, [r16 * 2]),
                                 jnp.int32)
                y = plsc.bitcast(plsc.load_gather(gbuf_v, [r16 * 2 + 1]),
                                 jnp.int32)
                cell_v[pl.ds(ci * (pchunk // 2) + v * 16, 16)] = jnp.where(
                    y < GY, x * GY + y, NROWS)
                return carry2

            return lax.fori_loop(0, pchunk // 32, cell_vec, carry)

        lax.fori_loop(0, 5, cell_chunk, 0)

        def zero_vec(r, carry):
            gbuf_v[pl.ds(r * 16, 16)] = zf
            return carry

        def fire(g):
            for hf, sem in ((0, sem0), (1, sem1)):
                pltpu.async_copy(
                    ze_hbm.at[idx_v.at[pl.ds(g * grp * k_nbr + hf * half,
                                             half)]],
                    ze_v.at[pl.ds(hf * half, half)], sem)

        def drain(hf, sem):
            pltpu.make_async_copy(
                ze_hbm.at[idx_v.at[pl.ds(hf * half, half)]],
                ze_v.at[pl.ds(hf * half, half)], sem).wait()

        fire(0)

        def body(g, carry):
            @pl.when(g > 0)
            def _():
                for q in range(4):
                    pltpu.make_async_copy(
                        gbuf_v.at[pl.ds(q * PLANE, GROWS * 128)],
                        out_hbm.at[pl.ds(q * (out_words // 4), GROWS * 128)],
                        osem).wait()

            lax.fori_loop(0, 4 * PLANE // 16, zero_vec, 0)
            for s8 in range(grp):
                if s8 == 0:
                    drain(0, sem0)
                elif s8 == grp // 2:
                    drain(1, sem1)
                i = g * grp + s8
                for h in range(k_nbr // 16):
                    idx16 = idx_v[pl.ds(i * k_nbr + h * 16, 16)]
                    c = plsc.load_gather(cell_v, [idx16])
                    for p, (dlt, lo, hi) in enumerate(
                            ((3, 0, NROWS - 4), (0, 0, NROWS - 1),
                             (-3, 3, NROWS - 1))):
                        r = c + dlt
                        t = (r * 456) >> 12
                        l512 = (r - 9 * t) * 48 + p * 16
                        q = l512 >> 7
                        row = jnp.where((c >= lo) & (c <= hi),
                                        s8 * 5 + t, GROWS)
                        off = q * PLANE + row * 128 + (l512 & 127)
                        for kk in range(16):
                            o = off[kk]
                            k = s8 * k_nbr + h * 16 + kk
                            gbuf_v[pl.ds(o, 16)] = (
                                ze_v[k, pl.ds(p * 16, 16)])
            @pl.when(g + 1 < ngrp)
            def _():
                fire(g + 1)

            for q in range(4):
                pltpu.async_copy(
                    gbuf_v.at[pl.ds(q * PLANE, GROWS * 128)],
                    out_hbm.at[pl.ds(q * (out_words // 4)
                                     + (base + g * grp) * 5 * 128,
                                     GROWS * 128)], osem)
            return carry

        lax.fori_loop(0, ngrp, body, 0)
        for q in range(4):
            pltpu.make_async_copy(
                gbuf_v.at[pl.ds(q * PLANE, GROWS * 128)],
                out_hbm.at[pl.ds(q * (out_words // 4), GROWS * 128)],
                osem).wait()

    return sc_grid


def _head_body(g_ref, t_ref, w1b_ref, b3_ref, b1_ref, fcw_ref, fcb_ref,
               out_ref):
    H1 = jnp.dot(g_ref[0], t_ref[0], preferred_element_type=jnp.float32)
    for q in range(1, 4):
        H1 = H1 + jnp.dot(g_ref[q], t_ref[q],
                          preferred_element_type=jnp.float32)
    H1 = _leaky(H1 + b3_ref[...])
    H2 = _leaky(jnp.dot(H1, w1b_ref[...],
                        preferred_element_type=jnp.float32) + b1_ref[...])
    M = H2[:, 0:32]
    for j in range(1, 9):
        M = jnp.maximum(M, H2[:, 32 * j:32 * j + 32])
    nr = M.shape[0]
    ns = nr // 5
    si = lax.broadcasted_iota(jnp.int32, (ns, nr), 0)
    ti = lax.broadcasted_iota(jnp.int32, (ns, nr), 1)
    F = jnp.concatenate(
        [jnp.dot(jnp.where(ti == 5 * si + p, 1.0, 0.0), M,
                 preferred_element_type=jnp.float32) for p in range(5)],
        axis=1)
    out_ref[...] = _leaky(jnp.dot(F, fcw_ref[...],
                                  preferred_element_type=jnp.float32)
                          + fcb_ref[...])


def _tap_select():
    t = np.zeros((4, 128, 144), np.float32)
    for j in range(9):
        for p in range(3):
            for o in range(16):
                l512 = 48 * j + 16 * p + o
                t[l512 // 128, l512 % 128, 16 * j + o] = 1.0
    return jnp.asarray(t)


def kernel(encoder_outputs, neighbor_indices, grid_positions, conv3x1_w,
           conv3x1_b, conv1x1_w, conv1x1_b, fc_w, fc_b):
    na, C, T = encoder_outputs.shape
    batch, k_nbr = neighbor_indices.shape

    w3flat = jnp.transpose(conv3x1_w[:, :, :, 0], (1, 2, 0)).reshape(C, 48)
    wbig = (jnp.zeros((C, T, CPAD), jnp.float32)
            .at[:, T - 1, :48].set(w3flat).reshape(C * T, CPAD))
    w1 = jnp.transpose(conv1x1_w[:, :, 0, 0], (1, 0))
    w1b = jnp.zeros((144, 288), jnp.float32)
    for j in range(9):
        w1b = w1b.at[16 * j:16 * j + 16, 32 * j:32 * j + 32].set(w1)
    b3rep = jnp.tile(conv3x1_b, 9).reshape(1, 144)
    b1rep = jnp.tile(conv1x1_b, 9).reshape(1, 288)
    fcw2 = jnp.transpose(fc_w.reshape(64, 32, 5), (2, 1, 0)).reshape(160, 64)
    enc2d = encoder_outputs.reshape(na, C * T)

    rb = 2000
    ze = pl.pallas_call(
        _premul_body,
        grid=(na // rb,),
        in_specs=[pl.BlockSpec((rb, C * T), lambda i: (i, 0)),
                  pl.BlockSpec((C * T, CPAD), lambda i: (0, 0))],
        out_specs=pl.BlockSpec((rb, CPAD), lambda i: (i, 0)),
        out_shape=jax.ShapeDtypeStruct((na, CPAD), jnp.float32),
    )(enc2d, wbig)

    posf = lax.bitcast_convert_type(grid_positions.reshape(-1), jnp.float32)
    grid_flat = _make_sc_grid(na, batch, k_nbr)(
        ze, neighbor_indices.reshape(-1), posf)
    grid4 = grid_flat.reshape(4, batch * 5, 128)

    sb = 128
    out = pl.pallas_call(
        _head_body,
        grid=(batch // sb,),
        in_specs=[pl.BlockSpec((4, sb * 5, 128), lambda i: (0, i, 0)),
                  pl.BlockSpec((4, 128, 144), lambda i: (0, 0, 0)),
                  pl.BlockSpec((144, 288), lambda i: (0, 0)),
                  pl.BlockSpec((1, 144), lambda i: (0, 0)),
                  pl.BlockSpec((1, 288), lambda i: (0, 0)),
                  pl.BlockSpec((160, 64), lambda i: (0, 0)),
                  pl.BlockSpec((1, 64), lambda i: (0, 0))],
        out_specs=pl.BlockSpec((sb, 64), lambda i: (i, 0)),
        out_shape=jax.ShapeDtypeStruct((batch, 64), jnp.float32),
    )(grid4, _tap_select(), w1b, b3rep, b1rep, fcw2, fc_b.reshape(1, 64))
    return out


# speedup vs baseline: 5.8781x; 1.6503x over previous
import functools

import jax
import jax.numpy as jnp
import numpy as np
from jax import lax
from jax.experimental import pallas as pl
from jax.experimental.pallas import tpu as pltpu
from jax.experimental.pallas import tpu_sc as plsc

GX, GY = 15, 3
NROWS = GX * GY
CPAD = 128
PLANE = 5248
GROWS = 40


def _leaky(v):
    return jnp.where(v >= 0, v, 0.1 * v)


def _premul_body(enc_ref, w_ref, out_ref):
    out_ref[...] = jnp.dot(enc_ref[...], w_ref[...],
                           preferred_element_type=jnp.float32)


def _make_sc_grid(num_agents, batch, k_nbr):
    info = plsc.get_sparse_core_info()
    nc, ns = info.num_cores, info.num_subcores
    nw = nc * ns
    spw = batch // nw
    grp = 8
    ngrp = spw // grp
    npos = 2 * num_agents
    pchunk = npos // 5
    half = grp * k_nbr // 2
    out_words = batch * 5 * 512
    mesh = plsc.VectorSubcoreMesh(core_axis_name="c", subcore_axis_name="s")

    @functools.partial(
        pl.kernel,
        mesh=mesh,
        compiler_params=pltpu.CompilerParams(needs_layout_passes=False,
                                             use_tc_tiling_on_sc=False),
        out_type=jax.ShapeDtypeStruct((out_words,), jnp.float32),
        scratch_types=[
            pltpu.VMEM((num_agents,), jnp.int32),
            pltpu.VMEM((spw * k_nbr,), jnp.int32),
            pltpu.VMEM((grp * k_nbr, CPAD), jnp.float32),
            pltpu.VMEM((4 * PLANE,), jnp.float32),
            pltpu.SemaphoreType.DMA,
            pltpu.SemaphoreType.DMA,
            pltpu.SemaphoreType.DMA,
        ],
    )
    def sc_grid(ze_hbm, nbr_hbm, posf_hbm, out_hbm,
                cell_v, idx_v, ze_v, gbuf_v, sem0, sem1, osem):
        wid = lax.axis_index("s") * nc + lax.axis_index("c")
        base = wid * spw
        pltpu.sync_copy(nbr_hbm.at[pl.ds(base * k_nbr, spw * k_nbr)], idx_v)
        lanes = lax.iota(jnp.int32, 16)
        zf = jnp.zeros((16,), jnp.float32)

        def cell_chunk(ci, carry):
            pltpu.sync_copy(posf_hbm.at[pl.ds(ci * pchunk, pchunk)],
                            gbuf_v.at[pl.ds(0, pchunk)])

            def cell_vec(v, carry2):
                r16 = v * 16 + lanes
                x = plsc.bitcast(plsc.load_gather(gbuf_v, [r16 * 2]),
                                 jnp.int32)
                y = plsc.bitcast(plsc.load_gather(gbuf_v, [r16 * 2 + 1]),
                                 jnp.int32)
                cell_v[pl.ds(ci * (pchunk // 2) + v * 16, 16)] = jnp.where(
                    y < GY, x * GY + y, NROWS)
                return carry2

            return lax.fori_loop(0, pchunk // 32, cell_vec, carry)

        lax.fori_loop(0, 5, cell_chunk, 0)

        def zero_vec(r, carry):
            gbuf_v[pl.ds(r * 16, 16)] = zf
            return carry

        def fire(g):
            for hf, sem in ((0, sem0), (1, sem1)):
                pltpu.async_copy(
                    ze_hbm.at[idx_v.at[pl.ds(g * grp * k_nbr + hf * half,
                                             half)]],
                    ze_v.at[pl.ds(hf * half, half)], sem)

        def drain(hf, sem):
            pltpu.make_async_copy(
                ze_hbm.at[idx_v.at[pl.ds(hf * half, half)]],
                ze_v.at[pl.ds(hf * half, half)], sem).wait()

        fire(0)

        def body(g, carry):
            @pl.when(g > 0)
            def _():
                for q in range(4):
                    pltpu.make_async_copy(
                        gbuf_v.at[pl.ds(q * PLANE, GROWS * 128)],
                        out_hbm.at[pl.ds(q * (out_words // 4), GROWS * 128)],
                        osem).wait()

            lax.fori_loop(0, 4 * PLANE // 16, zero_vec, 0)
            for s8 in range(grp):
                if s8 == 0:
                    drain(0, sem0)
                elif s8 == grp // 2:
                    drain(1, sem1)
                i = g * grp + s8
                for h in range(k_nbr // 16):
                    idx16 = idx_v[pl.ds(i * k_nbr + h * 16, 16)]
                    c = plsc.load_gather(cell_v, [idx16])
                    for p, (dlt, lo, hi) in enumerate(
                            ((3, 0, NROWS - 4), (0, 0, NROWS - 1),
                             (-3, 3, NROWS - 1))):
                        r = c + dlt
                        t = (r * 456) >> 12
                        l512 = (r - 9 * t) * 48 + p * 16
                        q = l512 >> 7
                        row = jnp.where((c >= lo) & (c <= hi),
                                        s8 * 5 + t, GROWS)
                        off = q * PLANE + row * 128 + (l512 & 127)
                        for kk in range(16):
                            o = off[kk]
                            k = s8 * k_nbr + h * 16 + kk
                            gbuf_v[pl.ds(o, 16)] = (
                                ze_v[k, pl.ds(p * 16, 16)])
            @pl.when(g + 1 < ngrp)
            def _():
                fire(g + 1)

            for q in range(4):
                pltpu.async_copy(
                    gbuf_v.at[pl.ds(q * PLANE, GROWS * 128)],
                    out_hbm.at[pl.ds(q * (out_words // 4)
                                     + (base + g * grp) * 5 * 128,
                                     GROWS * 128)], osem)
            return carry

        lax.fori_loop(0, ngrp, body, 0)
        for q in range(4):
            pltpu.make_async_copy(
                gbuf_v.at[pl.ds(q * PLANE, GROWS * 128)],
                out_hbm.at[pl.ds(q * (out_words // 4), GROWS * 128)],
                osem).wait()

    return sc_grid


def _head_body(g_ref, t_ref, w1b_ref, b3_ref, b1_ref, fcw_ref, fcb_ref,
               out_ref):
    H1 = jnp.dot(g_ref[0], t_ref[0], preferred_element_type=jnp.float32)
    for q in range(1, 4):
        H1 = H1 + jnp.dot(g_ref[q], t_ref[q],
                          preferred_element_type=jnp.float32)
    H1 = _leaky(H1 + b3_ref[...])
    H2 = _leaky(jnp.dot(H1, w1b_ref[...],
                        preferred_element_type=jnp.float32) + b1_ref[...])
    M = H2[:, 0:32]
    for j in range(1, 9):
        M = jnp.maximum(M, H2[:, 32 * j:32 * j + 32])
    nr = M.shape[0]
    ns = nr // 5
    si = lax.broadcasted_iota(jnp.int32, (ns, nr), 0)
    ti = lax.broadcasted_iota(jnp.int32, (ns, nr), 1)
    F = jnp.concatenate(
        [jnp.dot(jnp.where(ti == 5 * si + p, 1.0, 0.0), M,
                 preferred_element_type=jnp.float32) for p in range(5)],
        axis=1)
    out_ref[...] = _leaky(jnp.dot(F, fcw_ref[...],
                                  preferred_element_type=jnp.float32)
                          + fcb_ref[...])


def _tap_select():
    t = np.zeros((4, 128, 144), np.float32)
    for j in range(9):
        for p in range(3):
            for o in range(16):
                l512 = 48 * j + 16 * p + o
                t[l512 // 128, l512 % 128, 16 * j + o] = 1.0
    return jnp.asarray(t)


def kernel(encoder_outputs, neighbor_indices, grid_positions, conv3x1_w,
           conv3x1_b, conv1x1_w, conv1x1_b, fc_w, fc_b):
    na, C, T = encoder_outputs.shape
    batch, k_nbr = neighbor_indices.shape

    w3flat = jnp.transpose(conv3x1_w[:, :, :, 0], (1, 2, 0)).reshape(C, 48)
    wbig = jnp.zeros((C, CPAD), jnp.float32).at[:, :48].set(w3flat)
    w1 = jnp.transpose(conv1x1_w[:, :, 0, 0], (1, 0))
    w1b = jnp.zeros((144, 288), jnp.float32)
    for j in range(9):
        w1b = w1b.at[16 * j:16 * j + 16, 32 * j:32 * j + 32].set(w1)
    b3rep = jnp.tile(conv3x1_b, 9).reshape(1, 144)
    b1rep = jnp.tile(conv1x1_b, 9).reshape(1, 288)
    fcw2 = jnp.transpose(fc_w.reshape(64, 32, 5), (2, 1, 0)).reshape(160, 64)
    enc_last = encoder_outputs[:, :, T - 1]
    rb = 2000
    ze = pl.pallas_call(
        _premul_body,
        grid=(na // rb,),
        in_specs=[pl.BlockSpec((rb, C), lambda i: (i, 0)),
                  pl.BlockSpec((C, CPAD), lambda i: (0, 0))],
        out_specs=pl.BlockSpec((rb, CPAD), lambda i: (i, 0)),
        out_shape=jax.ShapeDtypeStruct((na, CPAD), jnp.float32),
    )(enc_last, wbig)

    posf = lax.bitcast_convert_type(grid_positions.reshape(-1), jnp.float32)
    grid_flat = _make_sc_grid(na, batch, k_nbr)(
        ze, neighbor_indices.reshape(-1), posf)
    grid4 = grid_flat.reshape(4, batch * 5, 128)

    sb = 128
    out = pl.pallas_call(
        _head_body,
        grid=(batch // sb,),
        in_specs=[pl.BlockSpec((4, sb * 5, 128), lambda i: (0, i, 0)),
                  pl.BlockSpec((4, 128, 144), lambda i: (0, 0, 0)),
                  pl.BlockSpec((144, 288), lambda i: (0, 0)),
                  pl.BlockSpec((1, 144), lambda i: (0, 0)),
                  pl.BlockSpec((1, 288), lambda i: (0, 0)),
                  pl.BlockSpec((160, 64), lambda i: (0, 0)),
                  pl.BlockSpec((1, 64), lambda i: (0, 0))],
        out_specs=pl.BlockSpec((sb, 64), lambda i: (i, 0)),
        out_shape=jax.ShapeDtypeStruct((batch, 64), jnp.float32),
    )(grid4, _tap_select(), w1b, b3rep, b1rep, fcw2, fc_b.reshape(1, 64))
    return out


# speedup vs baseline: 6.7081x; 1.1412x over previous
import functools

import jax
import jax.numpy as jnp
import numpy as np
from jax import lax
from jax.experimental import pallas as pl
from jax.experimental.pallas import tpu as pltpu
from jax.experimental.pallas import tpu_sc as plsc

GX, GY = 15, 3
NROWS = GX * GY
CPAD = 128
PLANE = 5248
GROWS = 40


def _leaky(v):
    return jnp.where(v >= 0, v, 0.1 * v)


def _premul_body(enc_ref, w_ref, out_ref):
    out_ref[...] = jnp.dot(enc_ref[...], w_ref[...],
                           preferred_element_type=jnp.float32)


def _make_sc_grid(num_agents, batch, k_nbr):
    info = plsc.get_sparse_core_info()
    nc, ns = info.num_cores, info.num_subcores
    nw = nc * ns
    spw = batch // nw
    grp = 8
    ngrp = spw // grp
    npos = 2 * num_agents
    pchunk = npos // 5
    half = grp * k_nbr // 2
    out_words = batch * 5 * 512
    mesh = plsc.VectorSubcoreMesh(core_axis_name="c", subcore_axis_name="s")

    @functools.partial(
        pl.kernel,
        mesh=mesh,
        compiler_params=pltpu.CompilerParams(needs_layout_passes=False,
                                             use_tc_tiling_on_sc=False),
        out_type=jax.ShapeDtypeStruct((out_words,), jnp.float32),
        scratch_types=[
            pltpu.VMEM((num_agents,), jnp.int32),
            pltpu.VMEM((spw * k_nbr,), jnp.int32),
            pltpu.VMEM((grp * k_nbr, CPAD), jnp.float32),
            pltpu.VMEM((4 * PLANE,), jnp.float32),
            pltpu.SemaphoreType.DMA,
            pltpu.SemaphoreType.DMA,
            pltpu.SemaphoreType.DMA,
        ],
    )
    def sc_grid(ze_hbm, nbr_hbm, posf_hbm, out_hbm,
                cell_v, idx_v, ze_v, gbuf_v, sem0, sem1, osem):
        wid = lax.axis_index("s") * nc + lax.axis_index("c")
        base = wid * spw
        pltpu.sync_copy(nbr_hbm.at[pl.ds(base * k_nbr, spw * k_nbr)], idx_v)
        lanes = lax.iota(jnp.int32, 16)
        zf = jnp.zeros((16,), jnp.float32)

        def cell_chunk(ci, carry):
            pltpu.sync_copy(posf_hbm.at[pl.ds(ci * pchunk, pchunk)],
                            gbuf_v.at[pl.ds(0, pchunk)])

            def cell_vec(v, carry2):
                for u in range(5):
                    r16 = v * 80 + u * 16 + lanes
                    x = plsc.bitcast(plsc.load_gather(gbuf_v, [r16 * 2]),
                                     jnp.int32)
                    y = plsc.bitcast(plsc.load_gather(gbuf_v, [r16 * 2 + 1]),
                                     jnp.int32)
                    cell_v[pl.ds(ci * (pchunk // 2) + v * 80 + u * 16,
                                 16)] = jnp.where(y < GY, x * GY + y, NROWS)
                return carry2

            return lax.fori_loop(0, pchunk // 160, cell_vec, carry)

        lax.fori_loop(0, 5, cell_chunk, 0)

        def zero_vec(r, carry):
            for u in range(8):
                gbuf_v[pl.ds(r * 128 + u * 16, 16)] = zf
            return carry

        def fire(g):
            for hf, sem in ((0, sem0), (1, sem1)):
                pltpu.async_copy(
                    ze_hbm.at[idx_v.at[pl.ds(g * grp * k_nbr + hf * half,
                                             half)]],
                    ze_v.at[pl.ds(hf * half, half)], sem)

        def drain(hf, sem):
            pltpu.make_async_copy(
                ze_hbm.at[idx_v.at[pl.ds(hf * half, half)]],
                ze_v.at[pl.ds(hf * half, half)], sem).wait()

        fire(0)

        def body(g, carry):
            @pl.when(g > 0)
            def _():
                for q in range(4):
                    pltpu.make_async_copy(
                        gbuf_v.at[pl.ds(q * PLANE, GROWS * 128)],
                        out_hbm.at[pl.ds(q * (out_words // 4), GROWS * 128)],
                        osem).wait()

            lax.fori_loop(0, 4 * PLANE // 128, zero_vec, 0)
            for s8 in range(grp):
                if s8 == 0:
                    drain(0, sem0)
                elif s8 == grp // 2:
                    drain(1, sem1)
                i = g * grp + s8
                for h in range(k_nbr // 16):
                    idx16 = idx_v[pl.ds(i * k_nbr + h * 16, 16)]
                    c = plsc.load_gather(cell_v, [idx16])
                    for p, (dlt, lo, hi) in enumerate(
                            ((3, 0, NROWS - 4), (0, 0, NROWS - 1),
                             (-3, 3, NROWS - 1))):
                        r = c + dlt
                        t = (r * 456) >> 12
                        l512 = (r - 9 * t) * 48 + p * 16
                        q = l512 >> 7
                        row = jnp.where((c >= lo) & (c <= hi),
                                        s8 * 5 + t, GROWS)
                        off = q * PLANE + row * 128 + (l512 & 127)
                        for kk in range(16):
                            o = off[kk]
                            k = s8 * k_nbr + h * 16 + kk
                            gbuf_v[pl.ds(o, 16)] = (
                                ze_v[k, pl.ds(p * 16, 16)])
            @pl.when(g + 1 < ngrp)
            def _():
                fire(g + 1)

            for q in range(4):
                pltpu.async_copy(
                    gbuf_v.at[pl.ds(q * PLANE, GROWS * 128)],
                    out_hbm.at[pl.ds(q * (out_words // 4)
                                     + (base + g * grp) * 5 * 128,
                                     GROWS * 128)], osem)
            return carry

        lax.fori_loop(0, ngrp, body, 0)
        for q in range(4):
            pltpu.make_async_copy(
                gbuf_v.at[pl.ds(q * PLANE, GROWS * 128)],
                out_hbm.at[pl.ds(q * (out_words // 4), GROWS * 128)],
                osem).wait()

    return sc_grid


def _head_body(g_ref, t_ref, w1b_ref, b3_ref, b1_ref, fcw_ref, fcb_ref,
               out_ref):
    H1 = jnp.dot(g_ref[0], t_ref[0], preferred_element_type=jnp.float32)
    for q in range(1, 4):
        H1 = H1 + jnp.dot(g_ref[q], t_ref[q],
                          preferred_element_type=jnp.float32)
    H1 = _leaky(H1 + b3_ref[...])
    H2 = _leaky(jnp.dot(H1, w1b_ref[...],
                        preferred_element_type=jnp.float32) + b1_ref[...])
    M = H2[:, 0:32]
    for j in range(1, 9):
        M = jnp.maximum(M, H2[:, 32 * j:32 * j + 32])
    nr = M.shape[0]
    ns = nr // 5
    si = lax.broadcasted_iota(jnp.int32, (ns, nr), 0)
    ti = lax.broadcasted_iota(jnp.int32, (ns, nr), 1)
    F = jnp.concatenate(
        [jnp.dot(jnp.where(ti == 5 * si + p, 1.0, 0.0), M,
                 preferred_element_type=jnp.float32) for p in range(5)],
        axis=1)
    out_ref[...] = _leaky(jnp.dot(F, fcw_ref[...],
                                  preferred_element_type=jnp.float32)
                          + fcb_ref[...])


def _tap_select():
    t = np.zeros((4, 128, 144), np.float32)
    for j in range(9):
        for p in range(3):
            for o in range(16):
                l512 = 48 * j + 16 * p + o
                t[l512 // 128, l512 % 128, 16 * j + o] = 1.0
    return jnp.asarray(t)


def kernel(encoder_outputs, neighbor_indices, grid_positions, conv3x1_w,
           conv3x1_b, conv1x1_w, conv1x1_b, fc_w, fc_b):
    na, C, T = encoder_outputs.shape
    batch, k_nbr = neighbor_indices.shape

    w3flat = jnp.transpose(conv3x1_w[:, :, :, 0], (1, 2, 0)).reshape(C, 48)
    wbig = jnp.zeros((C, CPAD), jnp.float32).at[:, :48].set(w3flat)
    w1 = jnp.transpose(conv1x1_w[:, :, 0, 0], (1, 0))
    w1b = jnp.zeros((144, 288), jnp.float32)
    for j in range(9):
        w1b = w1b.at[16 * j:16 * j + 16, 32 * j:32 * j + 32].set(w1)
    b3rep = jnp.tile(conv3x1_b, 9).reshape(1, 144)
    b1rep = jnp.tile(conv1x1_b, 9).reshape(1, 288)
    fcw2 = jnp.transpose(fc_w.reshape(64, 32, 5), (2, 1, 0)).reshape(160, 64)
    enc_last = encoder_outputs[:, :, T - 1]
    rb = 2000
    ze = pl.pallas_call(
        _premul_body,
        grid=(na // rb,),
        in_specs=[pl.BlockSpec((rb, C), lambda i: (i, 0)),
                  pl.BlockSpec((C, CPAD), lambda i: (0, 0))],
        out_specs=pl.BlockSpec((rb, CPAD), lambda i: (i, 0)),
        out_shape=jax.ShapeDtypeStruct((na, CPAD), jnp.float32),
    )(enc_last, wbig)

    posf = lax.bitcast_convert_type(grid_positions.reshape(-1), jnp.float32)
    grid_flat = _make_sc_grid(na, batch, k_nbr)(
        ze, neighbor_indices.reshape(-1), posf)
    grid4 = grid_flat.reshape(4, batch * 5, 128)

    sb = 128
    out = pl.pallas_call(
        _head_body,
        grid=(batch // sb,),
        in_specs=[pl.BlockSpec((4, sb * 5, 128), lambda i: (0, i, 0)),
                  pl.BlockSpec((4, 128, 144), lambda i: (0, 0, 0)),
                  pl.BlockSpec((144, 288), lambda i: (0, 0)),
                  pl.BlockSpec((1, 144), lambda i: (0, 0)),
                  pl.BlockSpec((1, 288), lambda i: (0, 0)),
                  pl.BlockSpec((160, 64), lambda i: (0, 0)),
                  pl.BlockSpec((1, 64), lambda i: (0, 0))],
        out_specs=pl.BlockSpec((sb, 64), lambda i: (i, 0)),
        out_shape=jax.ShapeDtypeStruct((batch, 64), jnp.float32),
    )(grid4, _tap_select(), w1b, b3rep, b1rep, fcw2, fc_b.reshape(1, 64))
    return out
